# baseline (speedup 1.0000x reference)
import functools

import jax
import jax.numpy as jnp
from jax import lax
from jax.experimental import pallas as pl
from jax.experimental.pallas import tpu as pltpu
from jax.experimental.pallas import tpu_sc as plsc

N = 10000
E = 320000
D = 128
R = 4
SEG = N * R
NC = 2
NS = 16
NW = NC * NS
L = 16


_MM_BLK = 1000


def _mm_body(x_ref, m_ref, o_ref):
    o_ref[...] = jnp.dot(x_ref[...].astype(jnp.bfloat16),
                         m_ref[...].astype(jnp.bfloat16),
                         preferred_element_type=jnp.float32)


def _make_y(x, m):
    return pl.pallas_call(
        _mm_body,
        grid=(N // _MM_BLK,),
        in_specs=[
            pl.BlockSpec((_MM_BLK, D), lambda i: (i, 0)),
            pl.BlockSpec((D, R * D), lambda i: (0, 0)),
        ],
        out_specs=pl.BlockSpec((_MM_BLK, R * D), lambda i: (i, 0)),
        out_shape=jax.ShapeDtypeStruct((N, R * D), jnp.float32),
    )(x, m)


_EPW2 = E // NS
_CE2 = 2000
_NCH2 = _EPW2 // _CE2
_RED_TILES = 10
_RSL = SEG // _RED_TILES

_sc_mesh = plsc.VectorSubcoreMesh(core_axis_name="c", subcore_axis_name="s")
_sc_params = pltpu.CompilerParams(needs_layout_passes=False)


@functools.partial(
    pl.kernel,
    out_type=(jax.ShapeDtypeStruct((E,), jnp.float32),
              jax.ShapeDtypeStruct((E,), jnp.int32)),
    mesh=_sc_mesh,
    scratch_types=[
        pltpu.VMEM((_CE2,), jnp.int32),
        pltpu.VMEM((_CE2,), jnp.int32),
        pltpu.VMEM((_CE2,), jnp.int32),
        pltpu.VMEM((_CE2,), jnp.int32),
        pltpu.VMEM((_CE2,), jnp.int32),
        pltpu.VMEM((_CE2,), jnp.int32),
        pltpu.VMEM((_CE2,), jnp.int32),
        pltpu.VMEM((_RSL,), jnp.int32),
        pltpu.VMEM((_RSL,), jnp.float32),
        pltpu.VMEM((SEG,), jnp.float32),
        pltpu.VMEM((_EPW2,), jnp.int32),
        pltpu.VMEM((_EPW2,), jnp.int32),
        pltpu.VMEM((_EPW2,), jnp.float32),
        pltpu.VMEM_SHARED((SEG,), jnp.int32),
        pltpu.VMEM_SHARED((SEG,), jnp.float32),
        pltpu.SemaphoreType.DMA,
        pltpu.SemaphoreType.DMA,
    ],
    compiler_params=_sc_params,
)
def _deg_kernel(src_hbm, dst_hbm, rel_hbm, w_hbm, gidx_hbm,
                srcA, dstA, relA, srcB, dstB, relB,
                onesc, histb, winvb, winv_v,
                seg_t, gidx_t, w_t, hist_sh, winv_sh, msemA, msemB):
    cid = lax.axis_index("c")
    sid = lax.axis_index("s")
    zeros16 = jnp.zeros((L,), jnp.int32)
    ones16 = jnp.ones((L,), jnp.int32)

    @pl.when(cid == 0)
    def _():
        ebase = pl.multiple_of(sid * _EPW2, 8)

        @pl.when(sid < _RED_TILES)
        def _():
            def zb(i, _):
                histb[pl.ds(i * L, L)] = zeros16
                return 0
            lax.fori_loop(0, _RSL // L, zb, 0)
            pltpu.sync_copy(histb, hist_sh.at[pl.ds(sid * _RSL, _RSL)])

        def ob(i, _):
            onesc[pl.ds(i * L, L)] = ones16
            return 0
        lax.fori_loop(0, _CE2 // L, ob, 0)
        plsc.subcore_barrier()

        META = ((srcA, dstA, relA, msemA), (srcB, dstB, relB, msemB))

        def meta(j, bufs):
            sc, dc, rc, ms = bufs
            base = pl.multiple_of(ebase + j * _CE2, 8)
            pltpu.async_copy(src_hbm.at[pl.ds(base, _CE2)], sc, ms)
            pltpu.async_copy(dst_hbm.at[pl.ds(base, _CE2)], dc, ms)
            pltpu.async_copy(rel_hbm.at[pl.ds(base, _CE2)], rc, ms)

        def wait_meta(j, bufs):
            sc, dc, rc, ms = bufs
            base = pl.multiple_of(ebase + j * _CE2, 8)
            pltpu.make_async_copy(src_hbm.at[pl.ds(base, _CE2)], sc, ms).wait()
            pltpu.make_async_copy(dst_hbm.at[pl.ds(base, _CE2)], dc, ms).wait()
            pltpu.make_async_copy(rel_hbm.at[pl.ds(base, _CE2)], rc, ms).wait()

        def process(j, bufs):
            sc, dc, rc, _ = bufs
            wait_meta(j, bufs)
            off = j * _CE2

            def sg(i, _):
                sl = pl.ds(i * L, L)
                osl = pl.ds(off + i * L, L)
                r = rc[sl]
                dd = dc[sl]
                seg_t[osl] = dd * R + r
                gidx_t[osl] = sc[sl] * R + r
                dc[sl] = dd * R + r
                return 0
            lax.fori_loop(0, _CE2 // L, sg, 0)
            pltpu.sync_copy(onesc, hist_sh.at[dc], add=True)

        meta(0, META[0])

        def chunk(t, _):
            j0 = 2 * t
            meta(j0 + 1, META[1])
            process(j0, META[0])

            @pl.when(j0 + 2 < _NCH2)
            def _():
                meta(j0 + 2, META[0])
            process(j0 + 1, META[1])
            return 0
        lax.fori_loop(0, _NCH2 // 2, chunk, 0)
        pltpu.sync_copy(gidx_t, gidx_hbm.at[pl.ds(ebase, _EPW2)])
        plsc.subcore_barrier()

        @pl.when(sid < _RED_TILES)
        def _():
            off = sid * _RSL
            pltpu.sync_copy(hist_sh.at[pl.ds(off, _RSL)], histb)

            def wb(i, _):
                sl = pl.ds(i * L, L)
                d = jnp.maximum(histb[sl].astype(jnp.float32), 1.0)
                winvb[sl] = 1.0 / d
                return 0
            lax.fori_loop(0, _RSL // L, wb, 0)
            pltpu.sync_copy(winvb, winv_sh.at[pl.ds(off, _RSL)])
        plsc.subcore_barrier()

        pltpu.sync_copy(winv_sh, winv_v)

        def wg(i, _):
            sl = pl.ds(i * L, L)
            w_t[sl] = plsc.load_gather(winv_v, [seg_t[sl]])
            return 0
        lax.fori_loop(0, _EPW2 // L, wg, 0)
        pltpu.sync_copy(w_t, w_hbm.at[pl.ds(ebase, _EPW2)])


_EPW = E // NW
_CE = 80
_NCH = _EPW // _CE
_ZT = 10
_ZR = N // _ZT


@functools.partial(
    pl.kernel,
    out_type=jax.ShapeDtypeStruct((NC, N, D), jnp.float32),
    mesh=_sc_mesh,
    scratch_types=[
        pltpu.VMEM((_CE, D), jnp.float32),
        pltpu.VMEM((_CE, D), jnp.float32),
        pltpu.VMEM((_CE, D), jnp.float32),
        pltpu.VMEM((_CE,), jnp.int32),
        pltpu.VMEM((_CE,), jnp.int32),
        pltpu.VMEM((_CE,), jnp.int32),
        pltpu.VMEM((_CE,), jnp.float32),
        pltpu.VMEM((_CE,), jnp.float32),
        pltpu.VMEM((_CE,), jnp.float32),
        pltpu.VMEM((_EPW,), jnp.int32),
        pltpu.VMEM_SHARED((N, D), jnp.float32),
        pltpu.SemaphoreType.DMA,
        pltpu.SemaphoreType.DMA,
        pltpu.SemaphoreType.DMA,
        pltpu.SemaphoreType.DMA,
        pltpu.SemaphoreType.DMA,
        pltpu.SemaphoreType.DMA,
        pltpu.SemaphoreType.DMA,
        pltpu.SemaphoreType.DMA,
        pltpu.SemaphoreType.DMA,
    ],
    compiler_params=_sc_params,
)
def _spmm_kernel(y_hbm, dst_hbm, gidx_hbm, w_hbm, out_hbm,
                 rows0, rows1, rows2, dst0, dst1, dst2, wc0, wc1, wc2,
                 gidx_t, acc,
                 gsem0, gsem1, gsem2, ssem0, ssem1, ssem2,
                 msem0, msem1, msem2):
    cid = lax.axis_index("c")
    sid = lax.axis_index("s")
    wid = sid * NC + cid
    zeros16 = jnp.zeros((L,), jnp.float32)

    ROWS = (rows0, rows1, rows2)
    DST = (dst0, dst1, dst2)
    WC = (wc0, wc1, wc2)
    GSEM = (gsem0, gsem1, gsem2)
    SSEM = (ssem0, ssem1, ssem2)
    MSEM = (msem0, msem1, msem2)

    def zb(i, _):
        for k in range(D // L):
            rows0[i, pl.ds(k * L, L)] = zeros16
        return 0
    lax.fori_loop(0, _CE, zb, 0)

    @pl.when(sid < _ZT)
    def _():
        r0 = sid * _ZR
        for q in range(_ZR // _CE):
            pltpu.sync_copy(rows0, acc.at[pl.ds(r0 + q * _CE, _CE)])
        if _ZR % _CE:
            rem = _ZR % _CE
            pltpu.sync_copy(rows0.at[pl.ds(0, rem)],
                            acc.at[pl.ds(r0 + _ZR - rem, rem)])

    ebase = pl.multiple_of(wid * _EPW, 8)
    pltpu.sync_copy(gidx_hbm.at[pl.ds(ebase, _EPW)], gidx_t)
    plsc.subcore_barrier()

    def gather(j, b):
        idx = gidx_t.at[pl.ds(j * _CE, _CE)]
        return pltpu.async_copy(y_hbm.at[idx], ROWS[b], GSEM[b])

    def meta(j, b):
        pltpu.async_copy(dst_hbm.at[pl.ds(ebase + j * _CE, _CE)],
                         DST[b], MSEM[b])
        pltpu.async_copy(w_hbm.at[pl.ds(ebase + j * _CE, _CE)],
                         WC[b], MSEM[b])

    def wait_inputs(j, b):
        idx = gidx_t.at[pl.ds(j * _CE, _CE)]
        pltpu.make_async_copy(y_hbm.at[idx], ROWS[b], GSEM[b]).wait()
        pltpu.make_async_copy(dst_hbm.at[pl.ds(ebase + j * _CE, _CE)],
                              DST[b], MSEM[b]).wait()
        pltpu.make_async_copy(w_hbm.at[pl.ds(ebase + j * _CE, _CE)],
                              WC[b], MSEM[b]).wait()

    def scale(b):
        rows = ROWS[b]
        wcb = WC[b]

        def sb(g, _):
            w16 = wcb[pl.ds(g * L, L)]
            for jj in range(L):
                w = w16[jj]
                row = g * L + jj
                for k in range(D // L):
                    sl = pl.ds(k * L, L)
                    rows[row, sl] = rows[row, sl] * w
            return 0
        lax.fori_loop(0, _CE // L, sb, 0)

    def scatter(b):
        return pltpu.async_copy(ROWS[b], acc.at[DST[b]], SSEM[b], add=True)

    def wait_scatter(b):
        pltpu.make_async_copy(ROWS[b], acc.at[DST[b]], SSEM[b]).wait()

    def step(j, b):
        wait_inputs(j, b)
        scale(b)
        scatter(b)

        @pl.when(j >= 1)
        def _():
            wait_scatter((b + 2) % 3)

        @pl.when(j + 2 < _NCH)
        def _():
            gather(j + 2, (b + 2) % 3)
            meta(j + 2, (b + 2) % 3)

    gather(0, 0)
    meta(0, 0)
    gather(1, 1)
    meta(1, 1)

    def triple(t, _):
        for b in range(3):
            step(3 * t + b, b)
        return 0
    lax.fori_loop(0, _NCH // 3, triple, 0)

    step(_NCH - 2, 0)
    step(_NCH - 1, 1)
    wait_scatter(1)

    plsc.subcore_barrier()

    @pl.when(sid < _ZT)
    def _():
        r0 = sid * _ZR
        pltpu.sync_copy(acc.at[pl.ds(r0, _ZR)],
                        out_hbm.at[cid, pl.ds(r0, _ZR)])


_EP_BLK = 1000


def _epi_body(p0_ref, p1_ref, x_ref, wl_ref, wp1_ref, wp2_ref, wt1_ref,
              wt2_ref, b1_ref, bp_ref, bt_ref, o_ref):
    f32 = jnp.float32
    bf = jnp.bfloat16
    xb = x_ref[...]
    xc = xb.astype(bf)
    h = (p0_ref[...] + p1_ref[...]
         + jnp.dot(xc, wl_ref[...].astype(bf), preferred_element_type=f32)
         + b1_ref[...])
    hidden = jax.nn.sigmoid(h)
    hc = hidden.astype(bf)
    proj = jnp.maximum(
        jnp.dot(hc, wp1_ref[...].astype(bf), preferred_element_type=f32)
        + jnp.dot(xc, wp2_ref[...].astype(bf), preferred_element_type=f32)
        + bp_ref[...], 0.0)
    gate = jax.nn.sigmoid(
        jnp.dot(hc, wt1_ref[...].astype(bf), preferred_element_type=f32)
        + jnp.dot(xc, wt2_ref[...].astype(bf), preferred_element_type=f32)
        + bt_ref[...])
    o_ref[...] = gate * proj + (1.0 - gate) * hidden


def _epilogue(p0, p1, x, wl, wp1, wp2, wt1, wt2, b1, bp, bt):
    row_spec = pl.BlockSpec((_EP_BLK, D), lambda i: (i, 0))
    w_spec = pl.BlockSpec((D, D), lambda i: (0, 0))
    b_spec = pl.BlockSpec((1, D), lambda i: (0, 0))
    return pl.pallas_call(
        _epi_body,
        grid=(N // _EP_BLK,),
        in_specs=[row_spec, row_spec, row_spec,
                  w_spec, w_spec, w_spec, w_spec, w_spec,
                  b_spec, b_spec, b_spec],
        out_specs=row_spec,
        out_shape=jax.ShapeDtypeStruct((N, D), jnp.float32),
    )(p0, p1, x, wl, wp1, wp2, wt1, wt2, b1, bp, bt)


def kernel(x, edge_index, edge_type, W_rel, b_rel, W_loop, b_loop,
           W_proj, b_proj, W_trans, b_trans):
    src = edge_index[0].astype(jnp.int32)
    dst = edge_index[1].astype(jnp.int32)
    rel = edge_type.astype(jnp.int32)

    m = W_rel.reshape(D, R, D).transpose(2, 1, 0).reshape(D, R * D)
    y = _make_y(x, m).reshape(SEG, D)

    w_e, gidx_e = _deg_kernel(src, dst, rel)
    parts = _spmm_kernel(y, dst, gidx_e, w_e)

    b1 = (b_rel + b_loop).reshape(1, D)
    out = _epilogue(
        parts[0], parts[1], x,
        W_loop.T,
        W_proj[:, :D].T, W_proj[:, D:].T,
        W_trans[:, :D].T, W_trans[:, D:].T,
        b1, b_proj.reshape(1, D), b_trans.reshape(1, D))
    return out


# speedup vs baseline: 1.0032x; 1.0032x over previous
import functools

import jax
import jax.numpy as jnp
from jax import lax
from jax.experimental import pallas as pl
from jax.experimental.pallas import tpu as pltpu
from jax.experimental.pallas import tpu_sc as plsc

N = 10000
E = 320000
D = 128
R = 4
SEG = N * R
NC = 2
NS = 16
NW = NC * NS
L = 16


_MM_BLK = 1000


def _mm_body(x_ref, m_ref, o_ref):
    o_ref[...] = jnp.dot(x_ref[...], m_ref[...],
                         preferred_element_type=jnp.float32)


def _make_y(x, m):
    return pl.pallas_call(
        _mm_body,
        grid=(N // _MM_BLK,),
        in_specs=[
            pl.BlockSpec((_MM_BLK, D), lambda i: (i, 0)),
            pl.BlockSpec((D, R * D), lambda i: (0, 0)),
        ],
        out_specs=pl.BlockSpec((_MM_BLK, R * D), lambda i: (i, 0)),
        out_shape=jax.ShapeDtypeStruct((N, R * D), jnp.float32),
    )(x, m)


_EPW2 = E // NS
_CE2 = 2000
_NCH2 = _EPW2 // _CE2
_RED_TILES = 10
_RSL = SEG // _RED_TILES

_sc_mesh = plsc.VectorSubcoreMesh(core_axis_name="c", subcore_axis_name="s")
_sc_params = pltpu.CompilerParams(needs_layout_passes=False)


@functools.partial(
    pl.kernel,
    out_type=(jax.ShapeDtypeStruct((E,), jnp.float32),
              jax.ShapeDtypeStruct((E,), jnp.int32)),
    mesh=_sc_mesh,
    scratch_types=[
        pltpu.VMEM((_CE2,), jnp.int32),
        pltpu.VMEM((_CE2,), jnp.int32),
        pltpu.VMEM((_CE2,), jnp.int32),
        pltpu.VMEM((_CE2,), jnp.int32),
        pltpu.VMEM((_CE2,), jnp.int32),
        pltpu.VMEM((_CE2,), jnp.int32),
        pltpu.VMEM((_CE2,), jnp.int32),
        pltpu.VMEM((_RSL,), jnp.int32),
        pltpu.VMEM((_RSL,), jnp.float32),
        pltpu.VMEM((SEG,), jnp.float32),
        pltpu.VMEM((_EPW2,), jnp.int32),
        pltpu.VMEM((_EPW2,), jnp.int32),
        pltpu.VMEM((_EPW2,), jnp.float32),
        pltpu.VMEM_SHARED((SEG,), jnp.int32),
        pltpu.VMEM_SHARED((SEG,), jnp.float32),
        pltpu.SemaphoreType.DMA,
        pltpu.SemaphoreType.DMA,
    ],
    compiler_params=_sc_params,
)
def _deg_kernel(src_hbm, dst_hbm, rel_hbm, w_hbm, gidx_hbm,
                srcA, dstA, relA, srcB, dstB, relB,
                onesc, histb, winvb, winv_v,
                seg_t, gidx_t, w_t, hist_sh, winv_sh, msemA, msemB):
    cid = lax.axis_index("c")
    sid = lax.axis_index("s")
    zeros16 = jnp.zeros((L,), jnp.int32)
    ones16 = jnp.ones((L,), jnp.int32)

    @pl.when(cid == 0)
    def _():
        ebase = pl.multiple_of(sid * _EPW2, 8)

        @pl.when(sid < _RED_TILES)
        def _():
            def zb(i, _):
                histb[pl.ds(i * L, L)] = zeros16
                return 0
            lax.fori_loop(0, _RSL // L, zb, 0)
            pltpu.sync_copy(histb, hist_sh.at[pl.ds(sid * _RSL, _RSL)])

        def ob(i, _):
            onesc[pl.ds(i * L, L)] = ones16
            return 0
        lax.fori_loop(0, _CE2 // L, ob, 0)
        plsc.subcore_barrier()

        META = ((srcA, dstA, relA, msemA), (srcB, dstB, relB, msemB))

        def meta(j, bufs):
            sc, dc, rc, ms = bufs
            base = pl.multiple_of(ebase + j * _CE2, 8)
            pltpu.async_copy(src_hbm.at[pl.ds(base, _CE2)], sc, ms)
            pltpu.async_copy(dst_hbm.at[pl.ds(base, _CE2)], dc, ms)
            pltpu.async_copy(rel_hbm.at[pl.ds(base, _CE2)], rc, ms)

        def wait_meta(j, bufs):
            sc, dc, rc, ms = bufs
            base = pl.multiple_of(ebase + j * _CE2, 8)
            pltpu.make_async_copy(src_hbm.at[pl.ds(base, _CE2)], sc, ms).wait()
            pltpu.make_async_copy(dst_hbm.at[pl.ds(base, _CE2)], dc, ms).wait()
            pltpu.make_async_copy(rel_hbm.at[pl.ds(base, _CE2)], rc, ms).wait()

        def process(j, bufs):
            sc, dc, rc, _ = bufs
            wait_meta(j, bufs)
            off = j * _CE2

            def sg(i, _):
                sl = pl.ds(i * L, L)
                osl = pl.ds(off + i * L, L)
                r = rc[sl]
                dd = dc[sl]
                seg_t[osl] = dd * R + r
                gidx_t[osl] = sc[sl] * R + r
                dc[sl] = dd * R + r
                return 0
            lax.fori_loop(0, _CE2 // L, sg, 0)
            pltpu.sync_copy(onesc, hist_sh.at[dc], add=True)

        meta(0, META[0])

        def chunk(t, _):
            j0 = 2 * t
            meta(j0 + 1, META[1])
            process(j0, META[0])

            @pl.when(j0 + 2 < _NCH2)
            def _():
                meta(j0 + 2, META[0])
            process(j0 + 1, META[1])
            return 0
        lax.fori_loop(0, _NCH2 // 2, chunk, 0)
        pltpu.sync_copy(gidx_t, gidx_hbm.at[pl.ds(ebase, _EPW2)])
        plsc.subcore_barrier()

        @pl.when(sid < _RED_TILES)
        def _():
            off = sid * _RSL
            pltpu.sync_copy(hist_sh.at[pl.ds(off, _RSL)], histb)

            def wb(i, _):
                sl = pl.ds(i * L, L)
                d = jnp.maximum(histb[sl].astype(jnp.float32), 1.0)
                winvb[sl] = 1.0 / d
                return 0
            lax.fori_loop(0, _RSL // L, wb, 0)
            pltpu.sync_copy(winvb, winv_sh.at[pl.ds(off, _RSL)])
        plsc.subcore_barrier()

        pltpu.sync_copy(winv_sh, winv_v)

        def wg(i, _):
            sl = pl.ds(i * L, L)
            w_t[sl] = plsc.load_gather(winv_v, [seg_t[sl]])
            return 0
        lax.fori_loop(0, _EPW2 // L, wg, 0)
        pltpu.sync_copy(w_t, w_hbm.at[pl.ds(ebase, _EPW2)])


_EPW = E // NW
_CE = 80
_NCH = _EPW // _CE
_ZT = 10
_ZR = N // _ZT


@functools.partial(
    pl.kernel,
    out_type=jax.ShapeDtypeStruct((NC, N, D), jnp.float32),
    mesh=_sc_mesh,
    scratch_types=[
        pltpu.VMEM((_CE, D), jnp.float32),
        pltpu.VMEM((_CE, D), jnp.float32),
        pltpu.VMEM((_CE, D), jnp.float32),
        pltpu.VMEM((_CE,), jnp.int32),
        pltpu.VMEM((_CE,), jnp.int32),
        pltpu.VMEM((_CE,), jnp.int32),
        pltpu.VMEM((_CE,), jnp.float32),
        pltpu.VMEM((_CE,), jnp.float32),
        pltpu.VMEM((_CE,), jnp.float32),
        pltpu.VMEM((_EPW,), jnp.int32),
        pltpu.VMEM_SHARED((N, D), jnp.float32),
        pltpu.SemaphoreType.DMA,
        pltpu.SemaphoreType.DMA,
        pltpu.SemaphoreType.DMA,
        pltpu.SemaphoreType.DMA,
        pltpu.SemaphoreType.DMA,
        pltpu.SemaphoreType.DMA,
        pltpu.SemaphoreType.DMA,
        pltpu.SemaphoreType.DMA,
        pltpu.SemaphoreType.DMA,
    ],
    compiler_params=_sc_params,
)
def _spmm_kernel(y_hbm, dst_hbm, gidx_hbm, w_hbm, out_hbm,
                 rows0, rows1, rows2, dst0, dst1, dst2, wc0, wc1, wc2,
                 gidx_t, acc,
                 gsem0, gsem1, gsem2, ssem0, ssem1, ssem2,
                 msem0, msem1, msem2):
    cid = lax.axis_index("c")
    sid = lax.axis_index("s")
    wid = sid * NC + cid
    zeros16 = jnp.zeros((L,), jnp.float32)

    ROWS = (rows0, rows1, rows2)
    DST = (dst0, dst1, dst2)
    WC = (wc0, wc1, wc2)
    GSEM = (gsem0, gsem1, gsem2)
    SSEM = (ssem0, ssem1, ssem2)
    MSEM = (msem0, msem1, msem2)

    def zb(i, _):
        for k in range(D // L):
            rows0[i, pl.ds(k * L, L)] = zeros16
        return 0
    lax.fori_loop(0, _CE, zb, 0)

    @pl.when(sid < _ZT)
    def _():
        r0 = sid * _ZR
        for q in range(_ZR // _CE):
            pltpu.sync_copy(rows0, acc.at[pl.ds(r0 + q * _CE, _CE)])
        if _ZR % _CE:
            rem = _ZR % _CE
            pltpu.sync_copy(rows0.at[pl.ds(0, rem)],
                            acc.at[pl.ds(r0 + _ZR - rem, rem)])

    ebase = pl.multiple_of(wid * _EPW, 8)
    pltpu.sync_copy(gidx_hbm.at[pl.ds(ebase, _EPW)], gidx_t)
    plsc.subcore_barrier()

    def gather(j, b):
        idx = gidx_t.at[pl.ds(j * _CE, _CE)]
        return pltpu.async_copy(y_hbm.at[idx], ROWS[b], GSEM[b])

    def meta(j, b):
        pltpu.async_copy(dst_hbm.at[pl.ds(ebase + j * _CE, _CE)],
                         DST[b], MSEM[b])
        pltpu.async_copy(w_hbm.at[pl.ds(ebase + j * _CE, _CE)],
                         WC[b], MSEM[b])

    def wait_inputs(j, b):
        idx = gidx_t.at[pl.ds(j * _CE, _CE)]
        pltpu.make_async_copy(y_hbm.at[idx], ROWS[b], GSEM[b]).wait()
        pltpu.make_async_copy(dst_hbm.at[pl.ds(ebase + j * _CE, _CE)],
                              DST[b], MSEM[b]).wait()
        pltpu.make_async_copy(w_hbm.at[pl.ds(ebase + j * _CE, _CE)],
                              WC[b], MSEM[b]).wait()

    def scale(b):
        rows = ROWS[b]
        wcb = WC[b]

        def sb(g, _):
            w16 = wcb[pl.ds(g * L, L)]
            for jj in range(L):
                w = w16[jj]
                row = g * L + jj
                for k in range(D // L):
                    sl = pl.ds(k * L, L)
                    rows[row, sl] = rows[row, sl] * w
            return 0
        lax.fori_loop(0, _CE // L, sb, 0)

    def scatter(b):
        return pltpu.async_copy(ROWS[b], acc.at[DST[b]], SSEM[b], add=True)

    def wait_scatter(b):
        pltpu.make_async_copy(ROWS[b], acc.at[DST[b]], SSEM[b]).wait()

    def step(j, b):
        wait_inputs(j, b)
        scale(b)
        scatter(b)

        @pl.when(j >= 1)
        def _():
            wait_scatter((b + 2) % 3)

        @pl.when(j + 2 < _NCH)
        def _():
            gather(j + 2, (b + 2) % 3)
            meta(j + 2, (b + 2) % 3)

    gather(0, 0)
    meta(0, 0)
    gather(1, 1)
    meta(1, 1)

    def triple(t, _):
        for b in range(3):
            step(3 * t + b, b)
        return 0
    lax.fori_loop(0, _NCH // 3, triple, 0)

    step(_NCH - 2, 0)
    step(_NCH - 1, 1)
    wait_scatter(1)

    plsc.subcore_barrier()

    @pl.when(sid < _ZT)
    def _():
        r0 = sid * _ZR
        pltpu.sync_copy(acc.at[pl.ds(r0, _ZR)],
                        out_hbm.at[cid, pl.ds(r0, _ZR)])


_EP_BLK = 1000


def _epi_body(p0_ref, p1_ref, x_ref, wl_ref, wp1_ref, wp2_ref, wt1_ref,
              wt2_ref, b1_ref, bp_ref, bt_ref, o_ref):
    xb = x_ref[...]
    h = (p0_ref[...] + p1_ref[...]
         + jnp.dot(xb, wl_ref[...], preferred_element_type=jnp.float32)
         + b1_ref[...])
    hidden = jax.nn.sigmoid(h)
    proj = jnp.maximum(
        jnp.dot(hidden, wp1_ref[...], preferred_element_type=jnp.float32)
        + jnp.dot(xb, wp2_ref[...], preferred_element_type=jnp.float32)
        + bp_ref[...], 0.0)
    gate = jax.nn.sigmoid(
        jnp.dot(hidden, wt1_ref[...], preferred_element_type=jnp.float32)
        + jnp.dot(xb, wt2_ref[...], preferred_element_type=jnp.float32)
        + bt_ref[...])
    o_ref[...] = gate * proj + (1.0 - gate) * hidden


def _epilogue(p0, p1, x, wl, wp1, wp2, wt1, wt2, b1, bp, bt):
    row_spec = pl.BlockSpec((_EP_BLK, D), lambda i: (i, 0))
    w_spec = pl.BlockSpec((D, D), lambda i: (0, 0))
    b_spec = pl.BlockSpec((1, D), lambda i: (0, 0))
    return pl.pallas_call(
        _epi_body,
        grid=(N // _EP_BLK,),
        in_specs=[row_spec, row_spec, row_spec,
                  w_spec, w_spec, w_spec, w_spec, w_spec,
                  b_spec, b_spec, b_spec],
        out_specs=row_spec,
        out_shape=jax.ShapeDtypeStruct((N, D), jnp.float32),
    )(p0, p1, x, wl, wp1, wp2, wt1, wt2, b1, bp, bt)


def kernel(x, edge_index, edge_type, W_rel, b_rel, W_loop, b_loop,
           W_proj, b_proj, W_trans, b_trans):
    src = edge_index[0].astype(jnp.int32)
    dst = edge_index[1].astype(jnp.int32)
    rel = edge_type.astype(jnp.int32)

    m = W_rel.reshape(D, R, D).transpose(2, 1, 0).reshape(D, R * D)
    y = _make_y(x, m).reshape(SEG, D)

    w_e, gidx_e = _deg_kernel(src, dst, rel)
    parts = _spmm_kernel(y, dst, gidx_e, w_e)

    b1 = (b_rel + b_loop).reshape(1, D)
    out = _epilogue(
        parts[0], parts[1], x,
        W_loop.T,
        W_proj[:, :D].T, W_proj[:, D:].T,
        W_trans[:, :D].T, W_trans[:, D:].T,
        b1, b_proj.reshape(1, D), b_trans.reshape(1, D))
    return out
